# baseline (speedup 1.0000x reference)
import functools

import jax
import jax.numpy as jnp
from jax import lax
from jax.experimental import pallas as pl
from jax.experimental.pallas import tpu as pltpu
from jax.experimental.pallas import tpu_sc as plsc

_NC = 2
_NS = 16
_NW = _NC * _NS
_CH = 128
_STAGES = 2


def _sc_gather(table, idx3, R, H):
    nchunk = idx3.shape[1]
    rows_per_w = nchunk * _CH
    mesh = plsc.VectorSubcoreMesh(
        core_axis_name="c", subcore_axis_name="s", num_cores=_NC, num_subcores=_NS
    )

    @functools.partial(
        pl.kernel,
        mesh=mesh,
        out_type=jax.ShapeDtypeStruct((R // 2, 2 * H), jnp.int32),
        compiler_params=pltpu.CompilerParams(use_tc_tiling_on_sc=False),
        scratch_types=[
            pltpu.VMEM((nchunk, _CH), jnp.int32),
            pltpu.VMEM((_CH, H), jnp.int32),
            pltpu.VMEM((_CH, H), jnp.int32),
            pltpu.SemaphoreType.DMA,
            pltpu.SemaphoreType.DMA,
        ],
    )
    def k(table_hbm, idx_hbm, out_hbm, idx_v, rows0, rows1, sem0, sem1):
        wid = lax.axis_index("s") * _NC + lax.axis_index("c")
        base = wid * (rows_per_w // 2)
        pltpu.sync_copy(idx_hbm.at[wid], idx_v)
        bufs = (rows0, rows1)
        sems = (sem0, sem1)
        dummy_src = table_hbm.at[pl.ds(0, _CH)]
        pltpu.make_async_copy(table_hbm.at[idx_v.at[0]], rows0, sem0).start()

        hc = _CH // 2

        def store(b, jj):
            o = base + jj * hc
            pltpu.sync_copy(
                bufs[b].at[pl.ds(0, hc)], out_hbm.at[pl.ds(o, hc), pl.ds(0, H)]
            )
            pltpu.sync_copy(
                bufs[b].at[pl.ds(hc, hc)],
                out_hbm.at[pl.ds(o, hc), pl.ds(H, H)],
            )

        def process(jj, b):
            pltpu.make_async_copy(
                table_hbm.at[idx_v.at[jj + 1]], bufs[1 - b], sems[1 - b]
            ).start()
            pltpu.make_async_copy(dummy_src, bufs[b], sems[b]).wait()
            store(b, jj)

        def body(j, _):
            process(j * 2, 0)
            process(j * 2 + 1, 1)
            return 0

        lax.fori_loop(0, nchunk // 2 - 1, body, 0, unroll=False)
        process(nchunk - 2, 0)
        pltpu.make_async_copy(dummy_src, bufs[1], sems[1]).wait()
        store(1, nchunk - 1)

    return k(table, idx3)


def _gelu_exact(v):
    return 0.5 * v * (1.0 + lax.erf(v * 0.7071067811865476))


def _tc_body(K, ef_ref, xj_ref, w1_ref, b1_ref, w2_ref, b2_ref, out_ref):
    w1 = w1_ref[...]
    w2 = w2_ref[...]
    b1 = b1_ref[...]
    b2 = b2_ref[...]
    nb = out_ref.shape[0]
    C = out_ref.shape[1]
    H = C // 2
    e = ef_ref[...]
    h = _gelu_exact(jnp.dot(e, w1, preferred_element_type=jnp.float32) + b1)
    f = _gelu_exact(jnp.dot(h, w2, preferred_element_type=jnp.float32) + b2)
    xi = xj_ref[...]
    x_lo = lax.bitcast_convert_type(xi << 16, jnp.float32)
    x_hi = lax.bitcast_convert_type(xi & jnp.int32(-65536), jnp.float32)
    x_even = jnp.concatenate([x_lo[:, :H], x_hi[:, :H]], axis=1)
    x_odd = jnp.concatenate([x_lo[:, H:], x_hi[:, H:]], axis=1)
    fr = f.reshape(nb * K // 2, 2, C)
    prod = fr[:, 0, :] * x_even + fr[:, 1, :] * x_odd
    out_ref[...] = jnp.sum(prod.reshape(nb, K // 2, C), axis=1)


def kernel(x, edge_features, E_idx, W1, b1, W2, b2):
    B, N, C = x.shape
    K = E_idx.shape[-1]
    R = B * N * K
    H = C // 2

    xb = x.reshape(B * N, C).astype(jnp.bfloat16)
    table = lax.bitcast_convert_type(
        jnp.stack([xb[:, :H], xb[:, H:]], axis=-1), jnp.int32
    )
    offs = (jnp.arange(B, dtype=jnp.int32) * N)[:, None, None]
    idx_flat = (
        (E_idx + offs)
        .reshape(R // _CH, _CH // 2, 2)
        .transpose(0, 2, 1)
        .reshape(R)
    )

    nodes_s = B * N // _STAGES
    rows_s = nodes_s * K
    nchunk = rows_s // (_NW * _CH)
    ef2 = edge_features.reshape(B * N * K, C)

    nb = 256
    blocks_s = nodes_s // nb

    w1t = W1.T
    w2t = W2.T
    b1r = b1.reshape(1, C)
    b2r = b2.reshape(1, C)
    idx4 = idx_flat.reshape(_STAGES, _NW, nchunk, _CH)
    outs = []
    for s in range(_STAGES):
        xj = _sc_gather(table, idx4[s], rows_s, H)
        tc = pl.pallas_call(
            functools.partial(_tc_body, K),
            grid=(blocks_s,),
            in_specs=[
                pl.BlockSpec((nb * K, C), lambda i, s=s: (s * blocks_s + i, 0)),
                pl.BlockSpec((nb * K // 2, C), lambda i: (i, 0)),
                pl.BlockSpec((C, C), lambda i: (0, 0)),
                pl.BlockSpec((1, C), lambda i: (0, 0)),
                pl.BlockSpec((C, C), lambda i: (0, 0)),
                pl.BlockSpec((1, C), lambda i: (0, 0)),
            ],
            out_specs=pl.BlockSpec((nb, C), lambda i: (i, 0)),
            out_shape=jax.ShapeDtypeStruct((nodes_s, C), jnp.float32),
        )
        outs.append(tc(ef2, xj, w1t, b1r, w2t, b2r))
    return jnp.concatenate(outs, axis=0).reshape(B, N, C)
